# baseline (speedup 1.0000x reference)
import functools

import jax
import jax.numpy as jnp
from jax import lax
from jax.experimental import pallas as pl
from jax.experimental.pallas import tpu as pltpu
from jax.experimental.pallas import tpu_sc as plsc

D = 64
BLK = 128
S = 50
NB1 = 16384
CPW = 4


@functools.lru_cache(maxsize=None)
def _build():
    info = plsc.get_sparse_core_info()
    NC = info.num_cores
    NW = NC * info.num_subcores
    n_blocks = S * CPW
    mesh = plsc.VectorSubcoreMesh(core_axis_name="c", subcore_axis_name="s")

    @functools.partial(
        pl.kernel,
        mesh=mesh,
        out_type=jax.ShapeDtypeStruct((S, D, NB1), jnp.float32),
        compiler_params=pltpu.CompilerParams(use_tc_tiling_on_sc=True,
                                             needs_layout_passes=False),
        scratch_types=[
            pltpu.VMEM((S, CPW * BLK), jnp.int32),
            pltpu.VMEM((2, BLK), jnp.int32),
            pltpu.VMEM((2, BLK, BLK), jnp.float32),
            pltpu.VMEM((2, D, BLK + 1), jnp.float32),
            pltpu.SemaphoreType.DMA,
            pltpu.SemaphoreType.DMA,
            pltpu.SemaphoreType.DMA,
            pltpu.SemaphoreType.DMA,
        ],
    )
    def gather_kernel(ids_hbm, tab_hbm, out_hbm, slab, pairs, block, stag,
                      sg0, sg1, sw0, sw1):
        sem_g = (sg0, sg1)
        sem_w = (sw0, sw1)
        wid = lax.axis_index("s") * NC + lax.axis_index("c")
        col0 = wid * (CPW * BLK)

        pltpu.sync_copy(ids_hbm.at[:, pl.ds(col0, CPW * BLK)], slab)

        iotas = [lax.iota(jnp.int32, 16) + 16 * g for g in range(8)]

        def compute_pairs(k, par):
            s = k >> 2
            off = (k & 3) * BLK
            for g in range(8):
                v = slab[s, pl.ds(off + 16 * g, 16)]
                pairs[par, pl.ds(16 * g, 16)] = v >> 1

        def issue_gather(par):
            pltpu.async_copy(tab_hbm.at[pairs.at[par]], block.at[par],
                             sem_g[par])

        def wait_gather(par):
            pltpu.make_async_copy(tab_hbm.at[pairs.at[par]], block.at[par],
                                  sem_g[par]).wait()

        def issue_write(k, par):
            s = k >> 2
            c = (k & 3) * BLK + col0
            pltpu.async_copy(stag.at[par, :, pl.ds(0, BLK)],
                             out_hbm.at[s, :, pl.ds(c, BLK)], sem_w[par])

        def wait_write(k, par):
            s = k >> 2
            c = (k & 3) * BLK + col0
            pltpu.make_async_copy(stag.at[par, :, pl.ds(0, BLK)],
                                  out_hbm.at[s, :, pl.ds(c, BLK)],
                                  sem_w[par]).wait()

        iota16 = lax.iota(jnp.int32, 16)
        dvecs = [iota16 + 16 * dd for dd in range(D // 16)]

        def transpose_block(k, par):
            s = k >> 2
            off = (k & 3) * BLK

            def tbody(g, carry):
                ids = slab[s, pl.ds(off + 16 * g, 16)]
                halves = (ids & 1) * D
                t0 = 16 * g
                for j in range(16):
                    t = t0 + j
                    half = halves[j]
                    tvec = jnp.full((16,), 1, jnp.int32) * t
                    for dd in range(D // 16):
                        val = block[par, t, pl.ds(half + 16 * dd, 16)]
                        plsc.store_scatter(stag.at[par], [dvecs[dd], tvec],
                                           val)
                return carry

            lax.fori_loop(0, BLK // 16, tbody, 0)

        for par in range(2):
            compute_pairs(par, par)
            issue_gather(par)

        def body(kk, carry):
            for par in range(2):
                k = 2 * kk + par
                wait_gather(par)

                @pl.when(k >= 2)
                def _():
                    wait_write(k - 2, par)

                transpose_block(k, par)
                issue_write(k, par)

                @pl.when(k < n_blocks - 2)
                def _():
                    compute_pairs(k + 2, par)
                    issue_gather(par)
            return carry

        lax.fori_loop(0, n_blocks // 2, body, 0)

        for par in range(2):
            wait_write(n_blocks - 2 + par, par)

    return gather_kernel


def kernel(token_ids, embedding_matrix):
    ids_t = token_ids.T.astype(jnp.int32)
    tab2 = embedding_matrix.reshape(500000, 128)
    out3 = _build()(ids_t, tab2)
    return jnp.transpose(out3, (2, 0, 1))


# speedup vs baseline: 1.0169x; 1.0169x over previous
import functools

import jax
import jax.numpy as jnp
from jax import lax
from jax.experimental import pallas as pl
from jax.experimental.pallas import tpu as pltpu
from jax.experimental.pallas import tpu_sc as plsc

D = 64
BLK = 128
S = 50
NB1 = 16384
CPW = 4


@functools.lru_cache(maxsize=None)
def _build():
    info = plsc.get_sparse_core_info()
    NC = info.num_cores
    NW = NC * info.num_subcores
    n_blocks = S * CPW
    mesh = plsc.VectorSubcoreMesh(core_axis_name="c", subcore_axis_name="s")

    @functools.partial(
        pl.kernel,
        mesh=mesh,
        out_type=jax.ShapeDtypeStruct((S, D, NB1), jnp.float32),
        compiler_params=pltpu.CompilerParams(use_tc_tiling_on_sc=True,
                                             needs_layout_passes=False),
        scratch_types=[
            pltpu.VMEM((S, CPW * BLK), jnp.int32),
            pltpu.VMEM((2, BLK), jnp.int32),
            pltpu.VMEM((2, BLK, BLK), jnp.float32),
            pltpu.VMEM((2, D, BLK + 1), jnp.float32),
            pltpu.SemaphoreType.DMA,
            pltpu.SemaphoreType.DMA,
            pltpu.SemaphoreType.DMA,
            pltpu.SemaphoreType.DMA,
        ],
    )
    def gather_kernel(ids_hbm, tab_hbm, out_hbm, slab, pairs, block, stag,
                      sg0, sg1, sw0, sw1):
        sem_g = (sg0, sg1)
        sem_w = (sw0, sw1)
        wid = lax.axis_index("s") * NC + lax.axis_index("c")
        col0 = wid * (CPW * BLK)

        pltpu.sync_copy(ids_hbm.at[:, pl.ds(col0, CPW * BLK)], slab)

        iotas = [lax.iota(jnp.int32, 16) + 16 * g for g in range(8)]

        def compute_pairs(k, par):
            s = k >> 2
            off = (k & 3) * BLK
            for g in range(8):
                v = slab[s, pl.ds(off + 16 * g, 16)]
                pairs[par, pl.ds(16 * g, 16)] = v >> 1

        def issue_gather(par):
            pltpu.async_copy(tab_hbm.at[pairs.at[par]], block.at[par],
                             sem_g[par])

        def wait_gather(par):
            pltpu.make_async_copy(tab_hbm.at[pairs.at[par]], block.at[par],
                                  sem_g[par]).wait()

        def issue_write(k, par):
            s = k >> 2
            c = (k & 3) * BLK + col0
            pltpu.async_copy(stag.at[par, :, pl.ds(0, BLK)],
                             out_hbm.at[s, :, pl.ds(c, BLK)], sem_w[par])

        def wait_write(k, par):
            s = k >> 2
            c = (k & 3) * BLK + col0
            pltpu.make_async_copy(stag.at[par, :, pl.ds(0, BLK)],
                                  out_hbm.at[s, :, pl.ds(c, BLK)],
                                  sem_w[par]).wait()

        iota16 = lax.iota(jnp.int32, 16)
        dvecs = [iota16 + 16 * dd for dd in range(D // 16)]

        def transpose_block(k, par):
            s = k >> 2
            off = (k & 3) * BLK

            nd = D // 16

            def tbody(g, carry):
                ids = slab[s, pl.ds(off + 16 * g, 16)]
                halves = (ids & 1) * D
                hs = [halves[j] for j in range(16)]
                t0 = 16 * g
                for j0 in range(0, 16, 2):
                    tv = [jnp.full((16,), 1, jnp.int32) * (t0 + j0 + jj)
                          for jj in range(2)]
                    vals = [block[par, t0 + j0 + jj,
                                  pl.ds(hs[j0 + jj] + 16 * dd, 16)]
                            for jj in range(2) for dd in range(nd)]
                    for jj in range(2):
                        for dd in range(nd):
                            plsc.store_scatter(stag.at[par],
                                               [dvecs[dd], tv[jj]],
                                               vals[jj * nd + dd])
                return carry

            lax.fori_loop(0, BLK // 16, tbody, 0)

        for par in range(2):
            compute_pairs(par, par)
            issue_gather(par)

        def body(kk, carry):
            for par in range(2):
                k = 2 * kk + par
                wait_gather(par)

                @pl.when(k >= 2)
                def _():
                    wait_write(k - 2, par)

                transpose_block(k, par)
                issue_write(k, par)

                @pl.when(k < n_blocks - 2)
                def _():
                    compute_pairs(k + 2, par)
                    issue_gather(par)
            return carry

        lax.fori_loop(0, n_blocks // 2, body, 0)

        for par in range(2):
            wait_write(n_blocks - 2 + par, par)

    return gather_kernel


def kernel(token_ids, embedding_matrix):
    ids_t = token_ids.T.astype(jnp.int32)
    tab2 = embedding_matrix.reshape(500000, 128)
    out3 = _build()(ids_t, tab2)
    return jnp.transpose(out3, (2, 0, 1))


# speedup vs baseline: 1.7960x; 1.7662x over previous
import functools

import jax
import jax.numpy as jnp
from jax import lax
from jax.experimental import pallas as pl
from jax.experimental.pallas import tpu as pltpu
from jax.experimental.pallas import tpu_sc as plsc

D = 64
BLK = 128
S = 50
NB1 = 16384
CPW = 4


@functools.lru_cache(maxsize=None)
def _build():
    info = plsc.get_sparse_core_info()
    NC = info.num_cores
    NW = NC * info.num_subcores
    n_blocks = S * CPW
    mesh = plsc.VectorSubcoreMesh(core_axis_name="c", subcore_axis_name="s")

    @functools.partial(
        pl.kernel,
        mesh=mesh,
        out_type=jax.ShapeDtypeStruct((S, D, NB1), jnp.float32),
        compiler_params=pltpu.CompilerParams(use_tc_tiling_on_sc=True,
                                             needs_layout_passes=False),
        scratch_types=[
            pltpu.VMEM((S, CPW * BLK), jnp.int32),
            pltpu.VMEM((2, BLK), jnp.int32),
            pltpu.VMEM((2, BLK, BLK), jnp.float32),
            pltpu.VMEM((D * (BLK + 1),), jnp.float32),
            pltpu.VMEM((D * (BLK + 1),), jnp.float32),
            pltpu.VMEM((2, D, BLK), jnp.float32),
            pltpu.SemaphoreType.DMA,
            pltpu.SemaphoreType.DMA,
            pltpu.SemaphoreType.DMA,
            pltpu.SemaphoreType.DMA,
        ],
    )
    def gather_kernel(ids_hbm, tab_hbm, out_hbm, slab, pairs, block, stagf0,
                      stagf1, stag, sg0, sg1, sw0, sw1):
        stagfs = (stagf0, stagf1)
        sem_g = (sg0, sg1)
        sem_w = (sw0, sw1)
        wid = lax.axis_index("s") * NC + lax.axis_index("c")
        col0 = wid * (CPW * BLK)

        pltpu.sync_copy(ids_hbm.at[:, pl.ds(col0, CPW * BLK)], slab)

        iotas = [lax.iota(jnp.int32, 16) + 16 * g for g in range(8)]

        def compute_pairs(k, par):
            s = k >> 2
            off = (k & 3) * BLK
            for g in range(8):
                v = slab[s, pl.ds(off + 16 * g, 16)]
                pairs[par, pl.ds(16 * g, 16)] = v >> 1

        def issue_gather(par):
            pltpu.async_copy(tab_hbm.at[pairs.at[par]], block.at[par],
                             sem_g[par])

        def wait_gather(par):
            pltpu.make_async_copy(tab_hbm.at[pairs.at[par]], block.at[par],
                                  sem_g[par]).wait()

        def issue_write(k, par):
            s = k >> 2
            c = (k & 3) * BLK + col0
            pltpu.async_copy(stag.at[par],
                             out_hbm.at[s, :, pl.ds(c, BLK)], sem_w[par])

        def wait_write(k, par):
            s = k >> 2
            c = (k & 3) * BLK + col0
            pltpu.make_async_copy(stag.at[par],
                                  out_hbm.at[s, :, pl.ds(c, BLK)],
                                  sem_w[par]).wait()

        iota16 = lax.iota(jnp.int32, 16)
        RW = BLK + 1
        dvecs = [(iota16 + 16 * dd) * RW
                 for dd in range(D // 16)]
        nd = D // 16

        def transpose_block(k, par):
            s = k >> 2
            off = (k & 3) * BLK

            def tbody(g, carry):
                ids = slab[s, pl.ds(off + 16 * g, 16)]
                halves = (ids & 1) * D
                hs = [halves[j] for j in range(16)]
                t0 = 16 * g
                for j0 in range(0, 16, 2):
                    vals = [block[par, t0 + j0 + jj,
                                  pl.ds(hs[j0 + jj] + 16 * dd, 16)]
                            for jj in range(2) for dd in range(nd)]
                    for jj in range(2):
                        for dd in range(nd):
                            plsc.store_scatter(stagfs[par],
                                               [dvecs[dd] + (t0 + j0 + jj)],
                                               vals[jj * nd + dd])
                return carry

            lax.fori_loop(0, BLK // 16, tbody, 0)

            def rbody(i, carry):
                for r in range(4):
                    d = i * 4 + r
                    vs = [stagfs[par][pl.ds(d * RW + 16 * q, 16)]
                          for q in range(8)]
                    for q in range(8):
                        stag[par, d, pl.ds(16 * q, 16)] = vs[q]
                return carry

            lax.fori_loop(0, D // 4, rbody, 0)

        for par in range(2):
            compute_pairs(par, par)
            issue_gather(par)

        def body(kk, carry):
            for par in range(2):
                k = 2 * kk + par
                wait_gather(par)

                @pl.when(k >= 2)
                def _():
                    wait_write(k - 2, par)

                transpose_block(k, par)
                issue_write(k, par)

                @pl.when(k < n_blocks - 2)
                def _():
                    compute_pairs(k + 2, par)
                    issue_gather(par)
            return carry

        lax.fori_loop(0, n_blocks // 2, body, 0)

        for par in range(2):
            wait_write(n_blocks - 2 + par, par)

    return gather_kernel


def kernel(token_ids, embedding_matrix):
    ids_t = token_ids.T.astype(jnp.int32)
    tab2 = embedding_matrix.reshape(500000, 128)
    out3 = _build()(ids_t, tab2)
    return jnp.transpose(out3, (2, 0, 1))


# speedup vs baseline: 2.6055x; 1.4507x over previous
import functools

import jax
import jax.numpy as jnp
from jax import lax
from jax.experimental import pallas as pl
from jax.experimental.pallas import tpu as pltpu
from jax.experimental.pallas import tpu_sc as plsc

D = 64
BLK = 128
S = 50
NB1 = 16384
CPW = 4
NUM_V = 1000000


VP = 128
NSTRIP = NUM_V // VP
PRW = D + 1


@functools.lru_cache(maxsize=None)
def _build_prep():
    info = plsc.get_sparse_core_info()
    NC = info.num_cores
    NW = NC * info.num_subcores
    base_strips = NSTRIP // NW
    extra = NSTRIP - base_strips * NW
    mesh = plsc.VectorSubcoreMesh(core_axis_name="c", subcore_axis_name="s")

    @functools.partial(
        pl.kernel,
        mesh=mesh,
        out_type=jax.ShapeDtypeStruct((NUM_V // 2, 2 * D), jnp.float32),
        compiler_params=pltpu.CompilerParams(use_tc_tiling_on_sc=True,
                                             needs_layout_passes=False),
        scratch_types=[
            pltpu.VMEM((2, D, VP), jnp.float32),
            pltpu.VMEM((VP * PRW,), jnp.float32),
            pltpu.VMEM((VP * PRW,), jnp.float32),
            pltpu.VMEM((2, VP // 2, 2 * D), jnp.float32),
            pltpu.SemaphoreType.DMA,
            pltpu.SemaphoreType.DMA,
            pltpu.SemaphoreType.DMA,
            pltpu.SemaphoreType.DMA,
        ],
    )
    def prep_kernel(tabt_hbm, tail_hbm, out_hbm, sbuf, pf0, pf1, sd,
                    si0, si1, so0, so1):
        pfs = (pf0, pf1)
        sem_i = (si0, si1)
        sem_o = (so0, so1)
        wid = lax.axis_index("s") * NC + lax.axis_index("c")
        nstrip = base_strips + jnp.where(wid < extra, 1, 0)
        s0 = wid * base_strips + jnp.minimum(wid, extra)

        iota16 = lax.iota(jnp.int32, 16)
        vvecs = [(iota16 + 16 * q) * PRW for q in range(VP // 16)]

        def strip_of(i):
            return s0 + i

        def issue_in(i, b):
            pltpu.async_copy(
                tabt_hbm.at[:, pl.ds(strip_of(i) * VP, VP)], sbuf.at[b],
                sem_i[b])

        def wait_in(i, b):
            pltpu.make_async_copy(
                tabt_hbm.at[:, pl.ds(strip_of(i) * VP, VP)], sbuf.at[b],
                sem_i[b]).wait()

        def issue_out(i, b):
            pltpu.async_copy(
                sd.at[b], out_hbm.at[pl.ds(strip_of(i) * (VP // 2), VP // 2),
                                     pl.ds(0, 2 * D)], sem_o[b])

        def wait_out(i, b):
            pltpu.make_async_copy(
                sd.at[b], out_hbm.at[pl.ds(strip_of(i) * (VP // 2), VP // 2),
                                     pl.ds(0, 2 * D)], sem_o[b]).wait()

        def transpose_strip(b, ncol):
            nq = ncol // 16

            def dbody(i, carry):
                for r in range(2):
                    d = i * 2 + r
                    vals = [sbuf[b, d, pl.ds(16 * q, 16)] for q in range(nq)]
                    for q in range(nq):
                        plsc.store_scatter(pfs[b], [vvecs[q] + d], vals[q])
                return carry

            lax.fori_loop(0, D // 2, dbody, 0)

            def rbody(i, carry):
                for r in range(2):
                    v = i * 2 + r
                    vals = [pfs[b][pl.ds(v * PRW + 16 * q, 16)]
                            for q in range(D // 16)]
                    for q in range(D // 16):
                        sd[b, v >> 1, pl.ds((v & 1) * D + 16 * q, 16)] = (
                            vals[q])
                return carry

            lax.fori_loop(0, ncol // 2, rbody, 0)

        for b in range(2):
            @pl.when(nstrip > b)
            def _():
                issue_in(b, b)

        def body(kk, carry):
            for b in range(2):
                i = 2 * kk + b

                @pl.when(i < nstrip)
                def _():
                    wait_in(i, b)

                    @pl.when(i >= 2)
                    def _():
                        wait_out(i - 2, b)

                    transpose_strip(b, VP)
                    issue_out(i, b)

                    @pl.when(i + 2 < nstrip)
                    def _():
                        issue_in(i + 2, b)
            return carry

        lax.fori_loop(0, (base_strips + 2) // 2, body, 0)

        for b in range(2):
            @pl.when(nstrip > b)
            def _():
                wait_out(nstrip - 2 + b, b)

        @pl.when(wid == NW - 1)
        def _():
            pltpu.sync_copy(tail_hbm, sd.at[0, pl.ds(0, 32), :])
            pltpu.sync_copy(sd.at[0, pl.ds(0, 32), :],
                            out_hbm.at[pl.ds(NSTRIP * (VP // 2), 32),
                                       pl.ds(0, 2 * D)])

    return prep_kernel


@functools.lru_cache(maxsize=None)
def _build():
    info = plsc.get_sparse_core_info()
    NC = info.num_cores
    NW = NC * info.num_subcores
    n_blocks = S * CPW
    mesh = plsc.VectorSubcoreMesh(core_axis_name="c", subcore_axis_name="s")

    @functools.partial(
        pl.kernel,
        mesh=mesh,
        out_type=jax.ShapeDtypeStruct((S, D, NB1), jnp.float32),
        compiler_params=pltpu.CompilerParams(use_tc_tiling_on_sc=True,
                                             needs_layout_passes=False),
        scratch_types=[
            pltpu.VMEM((S, CPW * BLK), jnp.int32),
            pltpu.VMEM((2, BLK), jnp.int32),
            pltpu.VMEM((2, BLK, BLK), jnp.float32),
            pltpu.VMEM((D * (BLK + 1),), jnp.float32),
            pltpu.VMEM((D * (BLK + 1),), jnp.float32),
            pltpu.VMEM((2, D, BLK), jnp.float32),
            pltpu.SemaphoreType.DMA,
            pltpu.SemaphoreType.DMA,
            pltpu.SemaphoreType.DMA,
            pltpu.SemaphoreType.DMA,
        ],
    )
    def gather_kernel(ids_hbm, tab_hbm, out_hbm, slab, pairs, block, stagf0,
                      stagf1, stag, sg0, sg1, sw0, sw1):
        stagfs = (stagf0, stagf1)
        sem_g = (sg0, sg1)
        sem_w = (sw0, sw1)
        wid = lax.axis_index("s") * NC + lax.axis_index("c")
        col0 = wid * (CPW * BLK)

        pltpu.sync_copy(ids_hbm.at[:, pl.ds(col0, CPW * BLK)], slab)

        iotas = [lax.iota(jnp.int32, 16) + 16 * g for g in range(8)]

        def compute_pairs(k, par):
            s = k >> 2
            off = (k & 3) * BLK
            for g in range(8):
                v = slab[s, pl.ds(off + 16 * g, 16)]
                pairs[par, pl.ds(16 * g, 16)] = v >> 1

        def issue_gather(par):
            pltpu.async_copy(tab_hbm.at[pairs.at[par]], block.at[par],
                             sem_g[par])

        def wait_gather(par):
            pltpu.make_async_copy(tab_hbm.at[pairs.at[par]], block.at[par],
                                  sem_g[par]).wait()

        def issue_write(k, par):
            s = k >> 2
            c = (k & 3) * BLK + col0
            pltpu.async_copy(stag.at[par],
                             out_hbm.at[s, :, pl.ds(c, BLK)], sem_w[par])

        def wait_write(k, par):
            s = k >> 2
            c = (k & 3) * BLK + col0
            pltpu.make_async_copy(stag.at[par],
                                  out_hbm.at[s, :, pl.ds(c, BLK)],
                                  sem_w[par]).wait()

        iota16 = lax.iota(jnp.int32, 16)
        RW = BLK + 1
        dvecs = [(iota16 + 16 * dd) * RW
                 for dd in range(D // 16)]
        nd = D // 16

        def transpose_block(k, par):
            s = k >> 2
            off = (k & 3) * BLK

            def tbody(g, carry):
                ids = slab[s, pl.ds(off + 16 * g, 16)]
                halves = (ids & 1) * D
                hs = [halves[j] for j in range(16)]
                t0 = 16 * g
                for j0 in range(0, 16, 2):
                    vals = [block[par, t0 + j0 + jj,
                                  pl.ds(hs[j0 + jj] + 16 * dd, 16)]
                            for jj in range(2) for dd in range(nd)]
                    for jj in range(2):
                        for dd in range(nd):
                            plsc.store_scatter(stagfs[par],
                                               [dvecs[dd] + (t0 + j0 + jj)],
                                               vals[jj * nd + dd])
                return carry

            lax.fori_loop(0, BLK // 16, tbody, 0)

            def rbody(i, carry):
                for r in range(4):
                    d = i * 4 + r
                    vs = [stagfs[par][pl.ds(d * RW + 16 * q, 16)]
                          for q in range(8)]
                    for q in range(8):
                        stag[par, d, pl.ds(16 * q, 16)] = vs[q]
                return carry

            lax.fori_loop(0, D // 4, rbody, 0)

        for par in range(2):
            compute_pairs(par, par)
            issue_gather(par)

        def body(kk, carry):
            for par in range(2):
                k = 2 * kk + par
                wait_gather(par)

                @pl.when(k >= 2)
                def _():
                    wait_write(k - 2, par)

                transpose_block(k, par)
                issue_write(k, par)

                @pl.when(k < n_blocks - 2)
                def _():
                    compute_pairs(k + 2, par)
                    issue_gather(par)
            return carry

        lax.fori_loop(0, n_blocks // 2, body, 0)

        for par in range(2):
            wait_write(n_blocks - 2 + par, par)

    return gather_kernel


def kernel(token_ids, embedding_matrix):
    ids_t = token_ids.T.astype(jnp.int32)
    tab_t = embedding_matrix.T
    tail = embedding_matrix[NSTRIP * VP:].reshape(32, 2 * D)
    tab2 = _build_prep()(tab_t, tail)
    out3 = _build()(ids_t, tab2)
    return jnp.transpose(out3, (2, 0, 1))
